# baseline (speedup 1.0000x reference)
import jax
import jax.numpy as jnp
from jax import lax
from jax.experimental import pallas as pl
from jax.experimental.pallas import tpu as pltpu
from jax.experimental.pallas import tpu_sc as plsc

D = 128
INV_SQRT_D = float(1.0 / (D ** 0.5))
NC = 2
NS = 16
W = NC * NS
CH = 80

N_NODE = 10000
N_TRI = 20000
E_SELF = 320000
E_CROSS = 60000
NCH_A = E_SELF // W // CH
NCH_B = E_SELF // NS // CH
CW = 30
SN_PAD = 10240
ST_PAD = 20480
BLKA = 5
NBLK_A = NCH_A // BLKA
BLKB = 10
NBLK_B = NCH_B // BLKB


def _ln(x, g, b):
    m = jnp.mean(x, axis=-1, keepdims=True)
    v = jnp.mean((x - m) ** 2, axis=-1, keepdims=True)
    return (x - m) / jnp.sqrt(v + 1e-5) * g + b


def _proj_body(x_ref, g_ref, b_ref, qw_ref, qb_ref, kw_ref, kb_ref, vw_ref, vb_ref,
               qo_ref, ko_ref, vo_ref):
    xn = _ln(x_ref[...], g_ref[...], b_ref[...])
    q = jnp.dot(xn, qw_ref[...], preferred_element_type=jnp.float32) + qb_ref[...]
    k = jnp.dot(xn, kw_ref[...], preferred_element_type=jnp.float32) + kb_ref[...]
    v = jnp.dot(xn, vw_ref[...], preferred_element_type=jnp.float32) + vb_ref[...]
    qo_ref[...] = q
    ko_ref[...] = k
    vo_ref[...] = v


def _proj(x, g, b, p, rb):
    n = x.shape[0]
    row = lambda i: (i, 0)
    zero = lambda i: (0, 0)
    full = lambda s: pl.BlockSpec(s, zero)
    return pl.pallas_call(
        _proj_body,
        grid=(n // rb,),
        in_specs=[pl.BlockSpec((rb, D), row), full((1, D)), full((1, D)),
                  full((D, D)), full((1, D)), full((D, D)), full((1, D)),
                  full((D, D)), full((1, D))],
        out_specs=[pl.BlockSpec((rb, D), row), pl.BlockSpec((rb, D), row),
                   pl.BlockSpec((rb, D), row)],
        out_shape=[jax.ShapeDtypeStruct((n, D), jnp.float32),
                   jax.ShapeDtypeStruct((n, D), jnp.float32),
                   jax.ShapeDtypeStruct((n, D), jnp.float32)],
    )(x, g.reshape(1, D), b.reshape(1, D),
      p['q_w'], p['q_b'].reshape(1, D),
      p['k_w'], p['k_b'].reshape(1, D),
      p['v_w'], p['v_b'].reshape(1, D))


def _sca_body(qn, kn, qt, kt, nsrc4, ndst4, tsrc4, tdst4, tndst4, ntdst4,
              pn_o, pt_o, ctn_o, cnt_o,
              qrows48, krows48, qrows32, krows32, onesv, zbuf, pblk, src_blk, dst_blk,
              ctn_sh, cnt_sh, semq1, semk1, semq2, semk2):
    c = lax.axis_index("c")
    s = lax.axis_index("s")
    w = s * NC + c
    lane = lax.iota(jnp.int32, 16)

    def zfill(i, _):
        zbuf[pl.ds(i * 16, 16)] = jnp.zeros((16,), jnp.float32)
        return 0
    lax.fori_loop(0, 40, zfill, 0)

    def ofill(i, _):
        onesv[pl.ds(i * 16, 16)] = jnp.ones((16,), jnp.float32)
        return 0
    lax.fori_loop(0, CH // 16, ofill, 0)

    pltpu.sync_copy(zbuf, ctn_sh.at[pl.ds(s * 640, 640)])
    for j in range(2):
        pltpu.sync_copy(zbuf, cnt_sh.at[pl.ds(s * 1280 + j * 640, 640)])
    plsc.subcore_barrier()

    def seg_job(q_hbm, k_hbm, src4_hbm, dst4_hbm, p_out):
        def blk(b, _):
            pltpu.sync_copy(src4_hbm.at[w, b], src_blk)
            pltpu.sync_copy(dst4_hbm.at[w, b], dst_blk)
            pltpu.async_copy(q_hbm.at[dst_blk.at[0, pl.ds(0, 48)]], qrows48, semq1)
            pltpu.async_copy(k_hbm.at[src_blk.at[0, pl.ds(0, 48)]], krows48, semk1)

            def chunk(ic, _):
                def do_groups(qr, kr, gs, ebase):
                    for g in gs:
                        vec = jnp.zeros((16,), jnp.float32)
                        for i in range(16):
                            r = g * 16 - ebase + i
                            acc = qr[r, pl.ds(0, 16)] * kr[r, pl.ds(0, 16)]
                            for jj in range(1, 8):
                                acc = acc + qr[r, pl.ds(16 * jj, 16)] * kr[r, pl.ds(16 * jj, 16)]
                            vec = jnp.where(lane == i, jnp.sum(acc), vec)
                        pblk[ic, pl.ds(g * 16, 16)] = jnp.exp(vec * INV_SQRT_D)

                pltpu.make_async_copy(q_hbm.at[dst_blk.at[ic, pl.ds(0, 48)]], qrows48, semq1).wait()
                pltpu.make_async_copy(k_hbm.at[src_blk.at[ic, pl.ds(0, 48)]], krows48, semk1).wait()
                pltpu.async_copy(q_hbm.at[dst_blk.at[ic, pl.ds(48, 32)]], qrows32, semq2)
                pltpu.async_copy(k_hbm.at[src_blk.at[ic, pl.ds(48, 32)]], krows32, semk2)
                do_groups(qrows48, krows48, (0, 1, 2), 0)

                @pl.when(ic < BLKA - 1)
                def _():
                    pltpu.async_copy(q_hbm.at[dst_blk.at[ic + 1, pl.ds(0, 48)]], qrows48, semq1)
                    pltpu.async_copy(k_hbm.at[src_blk.at[ic + 1, pl.ds(0, 48)]], krows48, semk1)

                pltpu.make_async_copy(q_hbm.at[dst_blk.at[ic, pl.ds(48, 32)]], qrows32, semq2).wait()
                pltpu.make_async_copy(k_hbm.at[src_blk.at[ic, pl.ds(48, 32)]], krows32, semk2).wait()
                do_groups(qrows32, krows32, (3, 4), 48)
                return 0
            lax.fori_loop(0, BLKA, chunk, 0)
            pltpu.sync_copy(pblk, p_out.at[w, b])
            return 0
        lax.fori_loop(0, NBLK_A, blk, 0)

    seg_job(qn, kn, nsrc4, ndst4, pn_o)
    seg_job(qt, kt, tsrc4, tdst4, pt_o)

    @pl.when(w < CW)
    def _():
        def cjob(dst4_hbm, sh):
            def cblk(bb, _):
                pltpu.sync_copy(dst4_hbm.at[w, bb], src_blk)

                def chunk(it, _):
                    pltpu.sync_copy(onesv, sh.at[src_blk.at[it]], add=True)
                    return 0
                lax.fori_loop(0, BLKA, chunk, 0)
                return 0
            lax.fori_loop(0, 5, cblk, 0)
        cjob(tndst4, ctn_sh)
        cjob(ntdst4, cnt_sh)

    plsc.subcore_barrier()
    pltpu.sync_copy(ctn_sh.at[pl.ds(s * 640, 640)], ctn_o.at[c, 0, pl.ds(s * 640, 640)])
    pltpu.sync_copy(cnt_sh.at[pl.ds(s * 1280, 1280)], cnt_o.at[c, 0, pl.ds(s * 1280, 1280)])


def _sca(qn, kn, qt, kt, nsrc4, ndst4, tsrc4, tdst4, tndst4, ntdst4):
    mesh = plsc.VectorSubcoreMesh(core_axis_name="c", subcore_axis_name="s")
    k = pl.kernel(
        _sca_body, mesh=mesh,
        compiler_params=pltpu.CompilerParams(needs_layout_passes=False),
        out_type=[jax.ShapeDtypeStruct((W, NBLK_A, BLKA, CH), jnp.float32),
                  jax.ShapeDtypeStruct((W, NBLK_A, BLKA, CH), jnp.float32),
                  jax.ShapeDtypeStruct((NC, 1, SN_PAD), jnp.float32),
                  jax.ShapeDtypeStruct((NC, 1, ST_PAD), jnp.float32)],
        scratch_types=[
            pltpu.VMEM((48, D), jnp.float32),
            pltpu.VMEM((48, D), jnp.float32),
            pltpu.VMEM((32, D), jnp.float32),
            pltpu.VMEM((32, D), jnp.float32),
            pltpu.VMEM((CH,), jnp.float32),
            pltpu.VMEM((640,), jnp.float32),
            pltpu.VMEM((BLKA, CH), jnp.float32),
            pltpu.VMEM((BLKA, CH), jnp.int32),
            pltpu.VMEM((BLKA, CH), jnp.int32),
            pltpu.VMEM_SHARED((SN_PAD,), jnp.float32),
            pltpu.VMEM_SHARED((ST_PAD,), jnp.float32),
            pltpu.SemaphoreType.DMA,
            pltpu.SemaphoreType.DMA,
            pltpu.SemaphoreType.DMA,
            pltpu.SemaphoreType.DMA,
        ],
    )
    return k(qn, kn, qt, kt, nsrc4, ndst4, tsrc4, tdst4, tndst4, ntdst4)


HALF_N = N_NODE // 2
HALF_T = N_TRI // 2
AC_N = 5120
AC_T = 10112


def _scb_body(vn, vt, nsrc4, ndst4, pn4, tsrc4, tdst4, pt4,
              accn_o, acct_o, sn_o, st_o,
              vrowsA, vrowsB, zbuf2, zbuf1, dstlocA, dstlocB,
              pblk, src_blk, dst_blk, acc_sh, s_sh, semA, semB, semSA, semSB):
    c = lax.axis_index("c")
    s = lax.axis_index("s")
    lane = lax.iota(jnp.int32, 16)

    def zfill(r, _):
        for jj in range(8):
            zbuf2[r, pl.ds(jj * 16, 16)] = jnp.zeros((16,), jnp.float32)
        return 0
    lax.fori_loop(0, 8, zfill, 0)

    def z1fill(i, _):
        zbuf1[pl.ds(i * 16, 16)] = jnp.zeros((16,), jnp.float32)
        return 0
    lax.fori_loop(0, 40, z1fill, 0)

    def zero_acc(rows_per_sub):
        def zcopy(q, _):
            pltpu.sync_copy(zbuf2, acc_sh.at[pl.ds(s * rows_per_sub + q * 8, 8)])
            return 0
        lax.fori_loop(0, rows_per_sub // 8, zcopy, 0)

    def zero_s(region):
        if region == AC_N:
            @pl.when(s < 8)
            def _():
                pltpu.sync_copy(zbuf1, s_sh.at[pl.ds(s * 640, 640)])
        else:
            @pl.when(s < 15)
            def _():
                pltpu.sync_copy(zbuf1, s_sh.at[pl.ds(s * 640, 640)])

            @pl.when(s == 15)
            def _():
                pltpu.sync_copy(zbuf1.at[pl.ds(0, 512)], s_sh.at[pl.ds(9600, 512)])

    def bjob(v_hbm, src4_hbm, dst4_hbm, p4_hbm, half, dummy):
        base = c * half

        def compute(row, vrows, dstloc):
            def group(g, _):
                dv = dst_blk[row, pl.ds(g * 16, 16)]
                loc = dv - base
                ok = (loc >= 0) & (loc < half)
                dstloc[pl.ds(g * 16, 16)] = jnp.where(ok, loc, dummy)
                pchunk = pblk[row, pl.ds(g * 16, 16)]
                for i in range(16):
                    e = g * 16 + i
                    pv = jnp.full((16,), pchunk[i], jnp.float32)
                    for jj in range(8):
                        vrows[e, pl.ds(jj * 16, 16)] = vrows[e, pl.ds(jj * 16, 16)] * pv
                return 0
            lax.fori_loop(0, CH // 16, group, 0)

        def super_it(t, _):
            pltpu.sync_copy(src4_hbm.at[s, t], src_blk)
            pltpu.sync_copy(dst4_hbm.at[s, t], dst_blk)
            pltpu.sync_copy(p4_hbm.at[s, t], pblk)
            for jp in range(BLKB // 2):
                ra, rb = 2 * jp, 2 * jp + 1
                if jp > 0:
                    pltpu.make_async_copy(vrowsA, acc_sh.at[dstlocA], semSA).wait()
                else:
                    @pl.when(t > 0)
                    def _():
                        pltpu.make_async_copy(vrowsA, acc_sh.at[dstlocA], semSA).wait()
                cpa = pltpu.async_copy(v_hbm.at[src_blk.at[ra]], vrowsA, semA)
                if jp > 0:
                    pltpu.make_async_copy(vrowsB, acc_sh.at[dstlocB], semSB).wait()
                else:
                    @pl.when(t > 0)
                    def _():
                        pltpu.make_async_copy(vrowsB, acc_sh.at[dstlocB], semSB).wait()
                cpb = pltpu.async_copy(v_hbm.at[src_blk.at[rb]], vrowsB, semB)
                cpa.wait()
                compute(ra, vrowsA, dstlocA)
                pltpu.sync_copy(pblk.at[ra], s_sh.at[dstlocA], add=True)
                pltpu.async_copy(vrowsA, acc_sh.at[dstlocA], semSA, add=True)
                cpb.wait()
                compute(rb, vrowsB, dstlocB)
                pltpu.sync_copy(pblk.at[rb], s_sh.at[dstlocB], add=True)
                pltpu.async_copy(vrowsB, acc_sh.at[dstlocB], semSB, add=True)
            return 0
        lax.fori_loop(0, NBLK_B, super_it, 0)
        pltpu.make_async_copy(vrowsA, acc_sh.at[dstlocA], semSA).wait()
        pltpu.make_async_copy(vrowsB, acc_sh.at[dstlocB], semSB).wait()

    zero_acc(AC_T // NS)
    zero_s(AC_T)
    plsc.subcore_barrier()
    bjob(vn, nsrc4, ndst4, pn4, HALF_N, HALF_N)
    plsc.subcore_barrier()
    pltpu.sync_copy(acc_sh.at[pl.ds(s * 320, 320)], accn_o.at[c, pl.ds(s * 320, 320)])

    @pl.when(s < 8)
    def _():
        pltpu.sync_copy(s_sh.at[pl.ds(s * 640, 640)], sn_o.at[c, 0, pl.ds(s * 640, 640)])
    plsc.subcore_barrier()

    zero_acc(AC_N // NS)
    zero_s(AC_N)
    plsc.subcore_barrier()
    bjob(vt, tsrc4, tdst4, pt4, HALF_T, HALF_T)
    plsc.subcore_barrier()
    pltpu.sync_copy(acc_sh.at[pl.ds(s * 632, 632)], acct_o.at[c, pl.ds(s * 632, 632)])

    @pl.when(s < 15)
    def _():
        pltpu.sync_copy(s_sh.at[pl.ds(s * 640, 640)], st_o.at[c, 0, pl.ds(s * 640, 640)])

    @pl.when(s == 15)
    def _():
        pltpu.sync_copy(s_sh.at[pl.ds(9600, 512)], st_o.at[c, 0, pl.ds(9600, 512)])


def _scb(vn, vt, nsrc4, ndst4, pn4, tsrc4, tdst4, pt4):
    mesh = plsc.VectorSubcoreMesh(core_axis_name="c", subcore_axis_name="s")
    k = pl.kernel(
        _scb_body, mesh=mesh,
        compiler_params=pltpu.CompilerParams(needs_layout_passes=False),
        out_type=[jax.ShapeDtypeStruct((NC, AC_N, D), jnp.float32),
                  jax.ShapeDtypeStruct((NC, AC_T, D), jnp.float32),
                  jax.ShapeDtypeStruct((NC, 1, AC_N), jnp.float32),
                  jax.ShapeDtypeStruct((NC, 1, AC_T), jnp.float32)],
        scratch_types=[
            pltpu.VMEM((CH, D), jnp.float32),
            pltpu.VMEM((CH, D), jnp.float32),
            pltpu.VMEM((8, D), jnp.float32),
            pltpu.VMEM((640,), jnp.float32),
            pltpu.VMEM((CH,), jnp.int32),
            pltpu.VMEM((CH,), jnp.int32),
            pltpu.VMEM((BLKB, CH), jnp.float32),
            pltpu.VMEM((BLKB, CH), jnp.int32),
            pltpu.VMEM((BLKB, CH), jnp.int32),
            pltpu.VMEM_SHARED((AC_T, D), jnp.float32),
            pltpu.VMEM_SHARED((AC_T,), jnp.float32),
            pltpu.SemaphoreType.DMA,
            pltpu.SemaphoreType.DMA,
            pltpu.SemaphoreType.DMA,
            pltpu.SemaphoreType.DMA,
        ],
    )
    return k(vn, vt, nsrc4, ndst4, pn4, tsrc4, tdst4, pt4)


def _epi_body(orig_ref, acc_ref, s_ref, msk_ref, gc_ref, bc_ref, vw_ref, vb_ref,
              ow_ref, ob_ref, gm_ref, bm_ref, w1_ref, b1_ref, w2_ref, b2_ref,
              out_ref):
    sa = acc_ref[...] / (s_ref[...] + 1e-16)
    x1 = orig_ref[...] + sa
    xn = _ln(x1, gc_ref[...], bc_ref[...])
    v = jnp.dot(xn, vw_ref[...], preferred_element_type=jnp.float32) + vb_ref[...]
    v = jnp.where(msk_ref[...] > 0, v, 0.0)
    xnew = jnp.dot(v, ow_ref[...], preferred_element_type=jnp.float32) + ob_ref[...] + xn
    x2 = x1 + xnew
    xm = _ln(x2, gm_ref[...], bm_ref[...])
    h = jax.nn.gelu(jnp.dot(xm, w1_ref[...], preferred_element_type=jnp.float32) + b1_ref[...])
    out_ref[...] = x2 + jnp.dot(h, w2_ref[...], preferred_element_type=jnp.float32) + b2_ref[...]


def _epi(orig, acc, sb, cb, gc, bc, pc, gm, bm, pm, rb):
    n = orig.shape[0]
    row = lambda i: (i, 0)
    zero = lambda i: (0, 0)
    full = lambda s: pl.BlockSpec(s, zero)
    return pl.pallas_call(
        _epi_body,
        grid=(n // rb,),
        in_specs=[pl.BlockSpec((rb, D), row), pl.BlockSpec((rb, D), row),
                  pl.BlockSpec((rb, D), row), pl.BlockSpec((rb, D), row),
                  full((1, D)), full((1, D)), full((D, D)), full((1, D)),
                  full((D, D)), full((1, D)), full((1, D)), full((1, D)),
                  full((D, 4 * D)), full((1, 4 * D)), full((4 * D, D)), full((1, D))],
        out_specs=pl.BlockSpec((rb, D), row),
        out_shape=jax.ShapeDtypeStruct((n, D), jnp.float32),
    )(orig, acc, sb, cb, gc.reshape(1, D), bc.reshape(1, D),
      pc['v_w'], pc['v_b'].reshape(1, D), pc['out_w'], pc['out_b'].reshape(1, D),
      gm.reshape(1, D), bm.reshape(1, D),
      pm['w1'], pm['b1'].reshape(1, 4 * D), pm['w2'], pm['b2'].reshape(1, D))


def kernel(node, triangle, params, node_edge_index, tri_edge_index, tn_edge_index, nt_edge_index):
    P = params
    ln = P['ln']

    nsrcA = node_edge_index[0].reshape(W, NBLK_A, BLKA, CH)
    ndstA = node_edge_index[1].reshape(W, NBLK_A, BLKA, CH)
    tsrcA = tri_edge_index[0].reshape(W, NBLK_A, BLKA, CH)
    tdstA = tri_edge_index[1].reshape(W, NBLK_A, BLKA, CH)
    nsrcB = node_edge_index[0].reshape(NS, NBLK_B, BLKB, CH)
    ndstB = node_edge_index[1].reshape(NS, NBLK_B, BLKB, CH)
    tsrcB = tri_edge_index[0].reshape(NS, NBLK_B, BLKB, CH)
    tdstB = tri_edge_index[1].reshape(NS, NBLK_B, BLKB, CH)
    tndst4 = tn_edge_index[1].reshape(CW, 5, BLKA, CH)
    ntdst4 = nt_edge_index[1].reshape(CW, 5, BLKA, CH)

    qn, kn, vn = _proj(node, ln['node_self_g'], ln['node_self_b'], P['node_sa'], 1000)
    qt, kt, vt = _proj(triangle, ln['tri_self_g'], ln['tri_self_b'], P['tri_sa'], 1000)

    pn4, pt4, ctn2, cnt2 = _sca(qn, kn, qt, kt, nsrcA, ndstA, tsrcA, tdstA, tndst4, ntdst4)

    accn, acct, sn2, st2 = _scb(vn, vt, nsrcB, ndstB, pn4.reshape(NS, NBLK_B, BLKB, CH),
                                tsrcB, tdstB, pt4.reshape(NS, NBLK_B, BLKB, CH))
    accn_full = jnp.concatenate([accn[0, :HALF_N], accn[1, :HALF_N]], axis=0)
    acct_full = jnp.concatenate([acct[0, :HALF_T], acct[1, :HALF_T]], axis=0)

    sn = jnp.concatenate([sn2[0, 0, :HALF_N], sn2[1, 0, :HALF_N]])
    st = jnp.concatenate([st2[0, 0, :HALF_T], st2[1, 0, :HALF_T]])
    ctn = ctn2[0, 0, :N_NODE] + ctn2[1, 0, :N_NODE]
    cnt = cnt2[0, 0, :N_TRI] + cnt2[1, 0, :N_TRI]

    node_out = _epi(node, accn_full,
                    jnp.broadcast_to(sn[:, None], (N_NODE, D)),
                    jnp.broadcast_to(ctn[:, None], (N_NODE, D)),
                    ln['node_cross_g'], ln['node_cross_b'], P['tri2node'],
                    ln['node_mlp_g'], ln['node_mlp_b'], P['mlp_node'], 1000)
    tri_out = _epi(triangle, acct_full,
                   jnp.broadcast_to(st[:, None], (N_TRI, D)),
                   jnp.broadcast_to(cnt[:, None], (N_TRI, D)),
                   ln['tri_cross_g'], ln['tri_cross_b'], P['node2tri'],
                   ln['tri_mlp_g'], ln['tri_mlp_b'], P['mlp_tri'], 1000)
    return node_out, tri_out


# speedup vs baseline: 1.3124x; 1.3124x over previous
import jax
import jax.numpy as jnp
from jax import lax
from jax.experimental import pallas as pl
from jax.experimental.pallas import tpu as pltpu
from jax.experimental.pallas import tpu_sc as plsc

D = 128
INV_SQRT_D = float(1.0 / (D ** 0.5))
NC = 2
NS = 16
W = NC * NS
CH = 80

N_NODE = 10000
N_TRI = 20000
E_SELF = 320000
E_CROSS = 60000
NCH_A = E_SELF // W // CH
NCH_B = E_SELF // NS // CH
CW = 30
SN_PAD = 10240
ST_PAD = 20480
BLKA = 5
NBLK_A = NCH_A // BLKA
BLKB = 10
NBLK_B = NCH_B // BLKB


def _ln(x, g, b):
    m = jnp.mean(x, axis=-1, keepdims=True)
    v = jnp.mean((x - m) ** 2, axis=-1, keepdims=True)
    return (x - m) / jnp.sqrt(v + 1e-5) * g + b


def _proj_body(x_ref, g_ref, b_ref, qw_ref, qb_ref, kw_ref, kb_ref, vw_ref, vb_ref,
               qo_ref, ko_ref, vo_ref):
    xn = _ln(x_ref[...], g_ref[...], b_ref[...])
    q = jnp.dot(xn, qw_ref[...], preferred_element_type=jnp.float32) + qb_ref[...]
    k = jnp.dot(xn, kw_ref[...], preferred_element_type=jnp.float32) + kb_ref[...]
    v = jnp.dot(xn, vw_ref[...], preferred_element_type=jnp.float32) + vb_ref[...]
    qo_ref[...] = q
    ko_ref[...] = k
    vo_ref[...] = v


def _proj(x, g, b, p, rb):
    n = x.shape[0]
    row = lambda i: (i, 0)
    zero = lambda i: (0, 0)
    full = lambda s: pl.BlockSpec(s, zero)
    return pl.pallas_call(
        _proj_body,
        grid=(n // rb,),
        in_specs=[pl.BlockSpec((rb, D), row), full((1, D)), full((1, D)),
                  full((D, D)), full((1, D)), full((D, D)), full((1, D)),
                  full((D, D)), full((1, D))],
        out_specs=[pl.BlockSpec((rb, D), row), pl.BlockSpec((rb, D), row),
                   pl.BlockSpec((rb, D), row)],
        out_shape=[jax.ShapeDtypeStruct((n, D), jnp.float32),
                   jax.ShapeDtypeStruct((n, D), jnp.float32),
                   jax.ShapeDtypeStruct((n, D), jnp.float32)],
    )(x, g.reshape(1, D), b.reshape(1, D),
      p['q_w'], p['q_b'].reshape(1, D),
      p['k_w'], p['k_b'].reshape(1, D),
      p['v_w'], p['v_b'].reshape(1, D))


def _sca_body(qn, kn, qt, kt, nsrc4, ndst4, tsrc4, tdst4, tndst4, ntdst4,
              pn_o, pt_o, ctn_o, cnt_o,
              qrows48, krows48, qrows32, krows32, onesv, zbuf, pblk, src_blk, dst_blk,
              ctn_sh, cnt_sh, semq1, semk1, semq2, semk2):
    c = lax.axis_index("c")
    s = lax.axis_index("s")
    w = s * NC + c
    lane = lax.iota(jnp.int32, 16)

    def zfill(i, _):
        zbuf[pl.ds(i * 16, 16)] = jnp.zeros((16,), jnp.float32)
        return 0
    lax.fori_loop(0, 40, zfill, 0)

    def ofill(i, _):
        onesv[pl.ds(i * 16, 16)] = jnp.ones((16,), jnp.float32)
        return 0
    lax.fori_loop(0, CH // 16, ofill, 0)

    pltpu.sync_copy(zbuf, ctn_sh.at[pl.ds(s * 640, 640)])
    for j in range(2):
        pltpu.sync_copy(zbuf, cnt_sh.at[pl.ds(s * 1280 + j * 640, 640)])
    plsc.subcore_barrier()

    def seg_job(q_hbm, k_hbm, src4_hbm, dst4_hbm, p_out):
        def blk(b, _):
            pltpu.sync_copy(src4_hbm.at[w, b], src_blk)
            pltpu.sync_copy(dst4_hbm.at[w, b], dst_blk)
            pltpu.async_copy(q_hbm.at[dst_blk.at[0, pl.ds(0, 48)]], qrows48, semq1)
            pltpu.async_copy(k_hbm.at[src_blk.at[0, pl.ds(0, 48)]], krows48, semk1)

            def chunk(ic, _):
                def do_groups(qr, kr, gs, ebase):
                    for g in gs:
                        def edge16(i, vec):
                            r = g * 16 - ebase + i
                            acc = qr[r, pl.ds(0, 16)] * kr[r, pl.ds(0, 16)]
                            for jj in range(1, 8):
                                acc = acc + qr[r, pl.ds(16 * jj, 16)] * kr[r, pl.ds(16 * jj, 16)]
                            return jnp.where(lane == i, jnp.sum(acc), vec)
                        vec = lax.fori_loop(0, 16, edge16, jnp.zeros((16,), jnp.float32))
                        pblk[ic, pl.ds(g * 16, 16)] = jnp.exp(vec * INV_SQRT_D)

                pltpu.make_async_copy(q_hbm.at[dst_blk.at[ic, pl.ds(0, 48)]], qrows48, semq1).wait()
                pltpu.make_async_copy(k_hbm.at[src_blk.at[ic, pl.ds(0, 48)]], krows48, semk1).wait()
                pltpu.async_copy(q_hbm.at[dst_blk.at[ic, pl.ds(48, 32)]], qrows32, semq2)
                pltpu.async_copy(k_hbm.at[src_blk.at[ic, pl.ds(48, 32)]], krows32, semk2)
                do_groups(qrows48, krows48, (0, 1, 2), 0)

                @pl.when(ic < BLKA - 1)
                def _():
                    pltpu.async_copy(q_hbm.at[dst_blk.at[ic + 1, pl.ds(0, 48)]], qrows48, semq1)
                    pltpu.async_copy(k_hbm.at[src_blk.at[ic + 1, pl.ds(0, 48)]], krows48, semk1)

                pltpu.make_async_copy(q_hbm.at[dst_blk.at[ic, pl.ds(48, 32)]], qrows32, semq2).wait()
                pltpu.make_async_copy(k_hbm.at[src_blk.at[ic, pl.ds(48, 32)]], krows32, semk2).wait()
                do_groups(qrows32, krows32, (3, 4), 48)
                return 0
            lax.fori_loop(0, BLKA, chunk, 0)
            pltpu.sync_copy(pblk, p_out.at[w, b])
            return 0
        lax.fori_loop(0, NBLK_A, blk, 0)

    seg_job(qn, kn, nsrc4, ndst4, pn_o)
    seg_job(qt, kt, tsrc4, tdst4, pt_o)

    @pl.when(w < CW)
    def _():
        def cjob(dst4_hbm, sh):
            def cblk(bb, _):
                pltpu.sync_copy(dst4_hbm.at[w, bb], src_blk)

                def chunk(it, _):
                    pltpu.sync_copy(onesv, sh.at[src_blk.at[it]], add=True)
                    return 0
                lax.fori_loop(0, BLKA, chunk, 0)
                return 0
            lax.fori_loop(0, 5, cblk, 0)
        cjob(tndst4, ctn_sh)
        cjob(ntdst4, cnt_sh)

    plsc.subcore_barrier()
    pltpu.sync_copy(ctn_sh.at[pl.ds(s * 640, 640)], ctn_o.at[c, 0, pl.ds(s * 640, 640)])
    pltpu.sync_copy(cnt_sh.at[pl.ds(s * 1280, 1280)], cnt_o.at[c, 0, pl.ds(s * 1280, 1280)])


def _sca(qn, kn, qt, kt, nsrc4, ndst4, tsrc4, tdst4, tndst4, ntdst4):
    mesh = plsc.VectorSubcoreMesh(core_axis_name="c", subcore_axis_name="s")
    k = pl.kernel(
        _sca_body, mesh=mesh,
        compiler_params=pltpu.CompilerParams(needs_layout_passes=False),
        out_type=[jax.ShapeDtypeStruct((W, NBLK_A, BLKA, CH), jnp.float32),
                  jax.ShapeDtypeStruct((W, NBLK_A, BLKA, CH), jnp.float32),
                  jax.ShapeDtypeStruct((NC, 1, SN_PAD), jnp.float32),
                  jax.ShapeDtypeStruct((NC, 1, ST_PAD), jnp.float32)],
        scratch_types=[
            pltpu.VMEM((48, D), jnp.float32),
            pltpu.VMEM((48, D), jnp.float32),
            pltpu.VMEM((32, D), jnp.float32),
            pltpu.VMEM((32, D), jnp.float32),
            pltpu.VMEM((CH,), jnp.float32),
            pltpu.VMEM((640,), jnp.float32),
            pltpu.VMEM((BLKA, CH), jnp.float32),
            pltpu.VMEM((BLKA, CH), jnp.int32),
            pltpu.VMEM((BLKA, CH), jnp.int32),
            pltpu.VMEM_SHARED((SN_PAD,), jnp.float32),
            pltpu.VMEM_SHARED((ST_PAD,), jnp.float32),
            pltpu.SemaphoreType.DMA,
            pltpu.SemaphoreType.DMA,
            pltpu.SemaphoreType.DMA,
            pltpu.SemaphoreType.DMA,
        ],
    )
    return k(qn, kn, qt, kt, nsrc4, ndst4, tsrc4, tdst4, tndst4, ntdst4)


HALF_N = N_NODE // 2
HALF_T = N_TRI // 2
AC_N = 5120
AC_T = 10112


def _scb_body(vn, vt, nsrc4, ndst4, pn4, tsrc4, tdst4, pt4,
              accn_o, acct_o, sn_o, st_o,
              vrowsA, vrowsB, zbuf2, zbuf1, dstlocA, dstlocB,
              pblk, src_blk, dst_blk, acc_sh, s_sh, semA, semB, semSA, semSB):
    c = lax.axis_index("c")
    s = lax.axis_index("s")
    lane = lax.iota(jnp.int32, 16)

    def zfill(r, _):
        for jj in range(8):
            zbuf2[r, pl.ds(jj * 16, 16)] = jnp.zeros((16,), jnp.float32)
        return 0
    lax.fori_loop(0, 8, zfill, 0)

    def z1fill(i, _):
        zbuf1[pl.ds(i * 16, 16)] = jnp.zeros((16,), jnp.float32)
        return 0
    lax.fori_loop(0, 40, z1fill, 0)

    def zero_acc(rows_per_sub):
        def zcopy(q, _):
            pltpu.sync_copy(zbuf2, acc_sh.at[pl.ds(s * rows_per_sub + q * 8, 8)])
            return 0
        lax.fori_loop(0, rows_per_sub // 8, zcopy, 0)

    def zero_s(region):
        if region == AC_N:
            @pl.when(s < 8)
            def _():
                pltpu.sync_copy(zbuf1, s_sh.at[pl.ds(s * 640, 640)])
        else:
            @pl.when(s < 15)
            def _():
                pltpu.sync_copy(zbuf1, s_sh.at[pl.ds(s * 640, 640)])

            @pl.when(s == 15)
            def _():
                pltpu.sync_copy(zbuf1.at[pl.ds(0, 512)], s_sh.at[pl.ds(9600, 512)])

    def bjob(v_hbm, src4_hbm, dst4_hbm, p4_hbm, half, dummy):
        base = c * half

        def compute(row, vrows, dstloc):
            def group(g, _):
                dv = dst_blk[row, pl.ds(g * 16, 16)]
                loc = dv - base
                ok = (loc >= 0) & (loc < half)
                dstloc[pl.ds(g * 16, 16)] = jnp.where(ok, loc, dummy)
                pchunk = pblk[row, pl.ds(g * 16, 16)]

                def edge16(i, _):
                    e = g * 16 + i
                    sc = jnp.sum(jnp.where(lane == i, pchunk, 0.0))
                    pv = jnp.full((16,), sc, jnp.float32)
                    for jj in range(8):
                        vrows[e, pl.ds(jj * 16, 16)] = vrows[e, pl.ds(jj * 16, 16)] * pv
                    return 0
                lax.fori_loop(0, 16, edge16, 0)
                return 0
            lax.fori_loop(0, CH // 16, group, 0)

        def super_it(t, _):
            pltpu.sync_copy(src4_hbm.at[s, t], src_blk)
            pltpu.sync_copy(dst4_hbm.at[s, t], dst_blk)
            pltpu.sync_copy(p4_hbm.at[s, t], pblk)
            for jp in range(BLKB // 2):
                ra, rb = 2 * jp, 2 * jp + 1
                if jp > 0:
                    pltpu.make_async_copy(vrowsA, acc_sh.at[dstlocA], semSA).wait()
                else:
                    @pl.when(t > 0)
                    def _():
                        pltpu.make_async_copy(vrowsA, acc_sh.at[dstlocA], semSA).wait()
                cpa = pltpu.async_copy(v_hbm.at[src_blk.at[ra]], vrowsA, semA)
                if jp > 0:
                    pltpu.make_async_copy(vrowsB, acc_sh.at[dstlocB], semSB).wait()
                else:
                    @pl.when(t > 0)
                    def _():
                        pltpu.make_async_copy(vrowsB, acc_sh.at[dstlocB], semSB).wait()
                cpb = pltpu.async_copy(v_hbm.at[src_blk.at[rb]], vrowsB, semB)
                cpa.wait()
                compute(ra, vrowsA, dstlocA)
                pltpu.sync_copy(pblk.at[ra], s_sh.at[dstlocA], add=True)
                pltpu.async_copy(vrowsA, acc_sh.at[dstlocA], semSA, add=True)
                cpb.wait()
                compute(rb, vrowsB, dstlocB)
                pltpu.sync_copy(pblk.at[rb], s_sh.at[dstlocB], add=True)
                pltpu.async_copy(vrowsB, acc_sh.at[dstlocB], semSB, add=True)
            return 0
        lax.fori_loop(0, NBLK_B, super_it, 0)
        pltpu.make_async_copy(vrowsA, acc_sh.at[dstlocA], semSA).wait()
        pltpu.make_async_copy(vrowsB, acc_sh.at[dstlocB], semSB).wait()

    zero_acc(AC_T // NS)
    zero_s(AC_T)
    plsc.subcore_barrier()
    bjob(vn, nsrc4, ndst4, pn4, HALF_N, HALF_N)
    plsc.subcore_barrier()
    pltpu.sync_copy(acc_sh.at[pl.ds(s * 320, 320)], accn_o.at[c, pl.ds(s * 320, 320)])

    @pl.when(s < 8)
    def _():
        pltpu.sync_copy(s_sh.at[pl.ds(s * 640, 640)], sn_o.at[c, 0, pl.ds(s * 640, 640)])
    plsc.subcore_barrier()

    zero_acc(AC_N // NS)
    zero_s(AC_N)
    plsc.subcore_barrier()
    bjob(vt, tsrc4, tdst4, pt4, HALF_T, HALF_T)
    plsc.subcore_barrier()
    pltpu.sync_copy(acc_sh.at[pl.ds(s * 632, 632)], acct_o.at[c, pl.ds(s * 632, 632)])

    @pl.when(s < 15)
    def _():
        pltpu.sync_copy(s_sh.at[pl.ds(s * 640, 640)], st_o.at[c, 0, pl.ds(s * 640, 640)])

    @pl.when(s == 15)
    def _():
        pltpu.sync_copy(s_sh.at[pl.ds(9600, 512)], st_o.at[c, 0, pl.ds(9600, 512)])


def _scb(vn, vt, nsrc4, ndst4, pn4, tsrc4, tdst4, pt4):
    mesh = plsc.VectorSubcoreMesh(core_axis_name="c", subcore_axis_name="s")
    k = pl.kernel(
        _scb_body, mesh=mesh,
        compiler_params=pltpu.CompilerParams(needs_layout_passes=False),
        out_type=[jax.ShapeDtypeStruct((NC, AC_N, D), jnp.float32),
                  jax.ShapeDtypeStruct((NC, AC_T, D), jnp.float32),
                  jax.ShapeDtypeStruct((NC, 1, AC_N), jnp.float32),
                  jax.ShapeDtypeStruct((NC, 1, AC_T), jnp.float32)],
        scratch_types=[
            pltpu.VMEM((CH, D), jnp.float32),
            pltpu.VMEM((CH, D), jnp.float32),
            pltpu.VMEM((8, D), jnp.float32),
            pltpu.VMEM((640,), jnp.float32),
            pltpu.VMEM((CH,), jnp.int32),
            pltpu.VMEM((CH,), jnp.int32),
            pltpu.VMEM((BLKB, CH), jnp.float32),
            pltpu.VMEM((BLKB, CH), jnp.int32),
            pltpu.VMEM((BLKB, CH), jnp.int32),
            pltpu.VMEM_SHARED((AC_T, D), jnp.float32),
            pltpu.VMEM_SHARED((AC_T,), jnp.float32),
            pltpu.SemaphoreType.DMA,
            pltpu.SemaphoreType.DMA,
            pltpu.SemaphoreType.DMA,
            pltpu.SemaphoreType.DMA,
        ],
    )
    return k(vn, vt, nsrc4, ndst4, pn4, tsrc4, tdst4, pt4)


def _epi_body(orig_ref, acc_ref, s_ref, msk_ref, gc_ref, bc_ref, vw_ref, vb_ref,
              ow_ref, ob_ref, gm_ref, bm_ref, w1_ref, b1_ref, w2_ref, b2_ref,
              out_ref):
    sa = acc_ref[...] / (s_ref[...] + 1e-16)
    x1 = orig_ref[...] + sa
    xn = _ln(x1, gc_ref[...], bc_ref[...])
    v = jnp.dot(xn, vw_ref[...], preferred_element_type=jnp.float32) + vb_ref[...]
    v = jnp.where(msk_ref[...] > 0, v, 0.0)
    xnew = jnp.dot(v, ow_ref[...], preferred_element_type=jnp.float32) + ob_ref[...] + xn
    x2 = x1 + xnew
    xm = _ln(x2, gm_ref[...], bm_ref[...])
    h = jax.nn.gelu(jnp.dot(xm, w1_ref[...], preferred_element_type=jnp.float32) + b1_ref[...])
    out_ref[...] = x2 + jnp.dot(h, w2_ref[...], preferred_element_type=jnp.float32) + b2_ref[...]


def _epi(orig, acc, sb, cb, gc, bc, pc, gm, bm, pm, rb):
    n = orig.shape[0]
    row = lambda i: (i, 0)
    zero = lambda i: (0, 0)
    full = lambda s: pl.BlockSpec(s, zero)
    return pl.pallas_call(
        _epi_body,
        grid=(n // rb,),
        in_specs=[pl.BlockSpec((rb, D), row), pl.BlockSpec((rb, D), row),
                  pl.BlockSpec((rb, D), row), pl.BlockSpec((rb, D), row),
                  full((1, D)), full((1, D)), full((D, D)), full((1, D)),
                  full((D, D)), full((1, D)), full((1, D)), full((1, D)),
                  full((D, 4 * D)), full((1, 4 * D)), full((4 * D, D)), full((1, D))],
        out_specs=pl.BlockSpec((rb, D), row),
        out_shape=jax.ShapeDtypeStruct((n, D), jnp.float32),
    )(orig, acc, sb, cb, gc.reshape(1, D), bc.reshape(1, D),
      pc['v_w'], pc['v_b'].reshape(1, D), pc['out_w'], pc['out_b'].reshape(1, D),
      gm.reshape(1, D), bm.reshape(1, D),
      pm['w1'], pm['b1'].reshape(1, 4 * D), pm['w2'], pm['b2'].reshape(1, D))


def kernel(node, triangle, params, node_edge_index, tri_edge_index, tn_edge_index, nt_edge_index):
    P = params
    ln = P['ln']

    nsrcA = node_edge_index[0].reshape(W, NBLK_A, BLKA, CH)
    ndstA = node_edge_index[1].reshape(W, NBLK_A, BLKA, CH)
    tsrcA = tri_edge_index[0].reshape(W, NBLK_A, BLKA, CH)
    tdstA = tri_edge_index[1].reshape(W, NBLK_A, BLKA, CH)
    nsrcB = node_edge_index[0].reshape(NS, NBLK_B, BLKB, CH)
    ndstB = node_edge_index[1].reshape(NS, NBLK_B, BLKB, CH)
    tsrcB = tri_edge_index[0].reshape(NS, NBLK_B, BLKB, CH)
    tdstB = tri_edge_index[1].reshape(NS, NBLK_B, BLKB, CH)
    tndst4 = tn_edge_index[1].reshape(CW, 5, BLKA, CH)
    ntdst4 = nt_edge_index[1].reshape(CW, 5, BLKA, CH)

    qn, kn, vn = _proj(node, ln['node_self_g'], ln['node_self_b'], P['node_sa'], 1000)
    qt, kt, vt = _proj(triangle, ln['tri_self_g'], ln['tri_self_b'], P['tri_sa'], 1000)

    pn4, pt4, ctn2, cnt2 = _sca(qn, kn, qt, kt, nsrcA, ndstA, tsrcA, tdstA, tndst4, ntdst4)

    accn, acct, sn2, st2 = _scb(vn, vt, nsrcB, ndstB, pn4.reshape(NS, NBLK_B, BLKB, CH),
                                tsrcB, tdstB, pt4.reshape(NS, NBLK_B, BLKB, CH))
    accn_full = jnp.concatenate([accn[0, :HALF_N], accn[1, :HALF_N]], axis=0)
    acct_full = jnp.concatenate([acct[0, :HALF_T], acct[1, :HALF_T]], axis=0)

    sn = jnp.concatenate([sn2[0, 0, :HALF_N], sn2[1, 0, :HALF_N]])
    st = jnp.concatenate([st2[0, 0, :HALF_T], st2[1, 0, :HALF_T]])
    ctn = ctn2[0, 0, :N_NODE] + ctn2[1, 0, :N_NODE]
    cnt = cnt2[0, 0, :N_TRI] + cnt2[1, 0, :N_TRI]

    node_out = _epi(node, accn_full,
                    jnp.broadcast_to(sn[:, None], (N_NODE, D)),
                    jnp.broadcast_to(ctn[:, None], (N_NODE, D)),
                    ln['node_cross_g'], ln['node_cross_b'], P['tri2node'],
                    ln['node_mlp_g'], ln['node_mlp_b'], P['mlp_node'], 1000)
    tri_out = _epi(triangle, acct_full,
                   jnp.broadcast_to(st[:, None], (N_TRI, D)),
                   jnp.broadcast_to(cnt[:, None], (N_TRI, D)),
                   ln['tri_cross_g'], ln['tri_cross_b'], P['node2tri'],
                   ln['tri_mlp_g'], ln['tri_mlp_b'], P['mlp_tri'], 1000)
    return node_out, tri_out


# speedup vs baseline: 1.3131x; 1.0005x over previous
import jax
import jax.numpy as jnp
from jax import lax
from jax.experimental import pallas as pl
from jax.experimental.pallas import tpu as pltpu
from jax.experimental.pallas import tpu_sc as plsc

D = 128
INV_SQRT_D = float(1.0 / (D ** 0.5))
NC = 2
NS = 16
W = NC * NS
CH = 80

N_NODE = 10000
N_TRI = 20000
E_SELF = 320000
E_CROSS = 60000
NCH_A = E_SELF // W // CH
NCH_B = E_SELF // NS // CH
CW = 30
SN_PAD = 10240
ST_PAD = 20480
BLKA = 5
NBLK_A = NCH_A // BLKA
BLKB = 10
NBLK_B = NCH_B // BLKB


def _ln(x, g, b):
    m = jnp.mean(x, axis=-1, keepdims=True)
    v = jnp.mean((x - m) ** 2, axis=-1, keepdims=True)
    return (x - m) / jnp.sqrt(v + 1e-5) * g + b


def _proj_body(x_ref, g_ref, b_ref, qw_ref, qb_ref, kw_ref, kb_ref, vw_ref, vb_ref,
               qo_ref, ko_ref, vo_ref):
    xn = _ln(x_ref[...], g_ref[...], b_ref[...])
    q = jnp.dot(xn, qw_ref[...], preferred_element_type=jnp.float32) + qb_ref[...]
    k = jnp.dot(xn, kw_ref[...], preferred_element_type=jnp.float32) + kb_ref[...]
    v = jnp.dot(xn, vw_ref[...], preferred_element_type=jnp.float32) + vb_ref[...]
    qo_ref[...] = q
    ko_ref[...] = k
    vo_ref[...] = v


def _proj(x, g, b, p, rb):
    n = x.shape[0]
    row = lambda i: (i, 0)
    zero = lambda i: (0, 0)
    full = lambda s: pl.BlockSpec(s, zero)
    return pl.pallas_call(
        _proj_body,
        grid=(n // rb,),
        in_specs=[pl.BlockSpec((rb, D), row), full((1, D)), full((1, D)),
                  full((D, D)), full((1, D)), full((D, D)), full((1, D)),
                  full((D, D)), full((1, D))],
        out_specs=[pl.BlockSpec((rb, D), row), pl.BlockSpec((rb, D), row),
                   pl.BlockSpec((rb, D), row)],
        out_shape=[jax.ShapeDtypeStruct((n, D), jnp.float32),
                   jax.ShapeDtypeStruct((n, D), jnp.float32),
                   jax.ShapeDtypeStruct((n, D), jnp.float32)],
    )(x, g.reshape(1, D), b.reshape(1, D),
      p['q_w'], p['q_b'].reshape(1, D),
      p['k_w'], p['k_b'].reshape(1, D),
      p['v_w'], p['v_b'].reshape(1, D))


def _sca_body(qn, kn, qt, kt, nsrc4, ndst4, tsrc4, tdst4, tndst4, ntdst4,
              pn_o, pt_o, ctn_o, cnt_o,
              qrows48, krows48, qrows32, krows32, onesv, zbuf, pblk, src_blk, dst_blk,
              ctn_sh, cnt_sh, semq1, semk1, semq2, semk2):
    c = lax.axis_index("c")
    s = lax.axis_index("s")
    w = s * NC + c
    lane = lax.iota(jnp.int32, 16)

    def zfill(i, _):
        zbuf[pl.ds(i * 16, 16)] = jnp.zeros((16,), jnp.float32)
        return 0
    lax.fori_loop(0, 40, zfill, 0)

    def ofill(i, _):
        onesv[pl.ds(i * 16, 16)] = jnp.ones((16,), jnp.float32)
        return 0
    lax.fori_loop(0, CH // 16, ofill, 0)

    pltpu.sync_copy(zbuf, ctn_sh.at[pl.ds(s * 640, 640)])
    for j in range(2):
        pltpu.sync_copy(zbuf, cnt_sh.at[pl.ds(s * 1280 + j * 640, 640)])
    plsc.subcore_barrier()

    def seg_job(q_hbm, k_hbm, src4_hbm, dst4_hbm, p_out):
        def blk(b, _):
            pltpu.sync_copy(src4_hbm.at[w, b], src_blk)
            pltpu.sync_copy(dst4_hbm.at[w, b], dst_blk)
            pltpu.async_copy(q_hbm.at[dst_blk.at[0, pl.ds(0, 48)]], qrows48, semq1)
            pltpu.async_copy(k_hbm.at[src_blk.at[0, pl.ds(0, 48)]], krows48, semk1)

            def chunk(ic, _):
                def do_groups(qr, kr, gs, ebase):
                    for g in gs:
                        def edge16(i, vec):
                            r = g * 16 - ebase + i
                            acc = qr[r, pl.ds(0, 16)] * kr[r, pl.ds(0, 16)]
                            for jj in range(1, 8):
                                acc = acc + qr[r, pl.ds(16 * jj, 16)] * kr[r, pl.ds(16 * jj, 16)]
                            return jnp.where(lane == i, jnp.sum(acc), vec)
                        vec = plsc.parallel_loop(0, 16, unroll=2,
                                                 carry=jnp.zeros((16,), jnp.float32))(edge16)
                        pblk[ic, pl.ds(g * 16, 16)] = jnp.exp(vec * INV_SQRT_D)

                pltpu.make_async_copy(q_hbm.at[dst_blk.at[ic, pl.ds(0, 48)]], qrows48, semq1).wait()
                pltpu.make_async_copy(k_hbm.at[src_blk.at[ic, pl.ds(0, 48)]], krows48, semk1).wait()
                pltpu.async_copy(q_hbm.at[dst_blk.at[ic, pl.ds(48, 32)]], qrows32, semq2)
                pltpu.async_copy(k_hbm.at[src_blk.at[ic, pl.ds(48, 32)]], krows32, semk2)
                do_groups(qrows48, krows48, (0, 1, 2), 0)

                @pl.when(ic < BLKA - 1)
                def _():
                    pltpu.async_copy(q_hbm.at[dst_blk.at[ic + 1, pl.ds(0, 48)]], qrows48, semq1)
                    pltpu.async_copy(k_hbm.at[src_blk.at[ic + 1, pl.ds(0, 48)]], krows48, semk1)

                pltpu.make_async_copy(q_hbm.at[dst_blk.at[ic, pl.ds(48, 32)]], qrows32, semq2).wait()
                pltpu.make_async_copy(k_hbm.at[src_blk.at[ic, pl.ds(48, 32)]], krows32, semk2).wait()
                do_groups(qrows32, krows32, (3, 4), 48)
                return 0
            lax.fori_loop(0, BLKA, chunk, 0)
            pltpu.sync_copy(pblk, p_out.at[w, b])
            return 0
        lax.fori_loop(0, NBLK_A, blk, 0)

    seg_job(qn, kn, nsrc4, ndst4, pn_o)
    seg_job(qt, kt, tsrc4, tdst4, pt_o)

    @pl.when(w < CW)
    def _():
        def cjob(dst4_hbm, sh):
            def cblk(bb, _):
                pltpu.sync_copy(dst4_hbm.at[w, bb], src_blk)

                def chunk(it, _):
                    pltpu.sync_copy(onesv, sh.at[src_blk.at[it]], add=True)
                    return 0
                lax.fori_loop(0, BLKA, chunk, 0)
                return 0
            lax.fori_loop(0, 5, cblk, 0)
        cjob(tndst4, ctn_sh)
        cjob(ntdst4, cnt_sh)

    plsc.subcore_barrier()
    pltpu.sync_copy(ctn_sh.at[pl.ds(s * 640, 640)], ctn_o.at[c, 0, pl.ds(s * 640, 640)])
    pltpu.sync_copy(cnt_sh.at[pl.ds(s * 1280, 1280)], cnt_o.at[c, 0, pl.ds(s * 1280, 1280)])


def _sca(qn, kn, qt, kt, nsrc4, ndst4, tsrc4, tdst4, tndst4, ntdst4):
    mesh = plsc.VectorSubcoreMesh(core_axis_name="c", subcore_axis_name="s")
    k = pl.kernel(
        _sca_body, mesh=mesh,
        compiler_params=pltpu.CompilerParams(needs_layout_passes=False),
        out_type=[jax.ShapeDtypeStruct((W, NBLK_A, BLKA, CH), jnp.float32),
                  jax.ShapeDtypeStruct((W, NBLK_A, BLKA, CH), jnp.float32),
                  jax.ShapeDtypeStruct((NC, 1, SN_PAD), jnp.float32),
                  jax.ShapeDtypeStruct((NC, 1, ST_PAD), jnp.float32)],
        scratch_types=[
            pltpu.VMEM((48, D), jnp.float32),
            pltpu.VMEM((48, D), jnp.float32),
            pltpu.VMEM((32, D), jnp.float32),
            pltpu.VMEM((32, D), jnp.float32),
            pltpu.VMEM((CH,), jnp.float32),
            pltpu.VMEM((640,), jnp.float32),
            pltpu.VMEM((BLKA, CH), jnp.float32),
            pltpu.VMEM((BLKA, CH), jnp.int32),
            pltpu.VMEM((BLKA, CH), jnp.int32),
            pltpu.VMEM_SHARED((SN_PAD,), jnp.float32),
            pltpu.VMEM_SHARED((ST_PAD,), jnp.float32),
            pltpu.SemaphoreType.DMA,
            pltpu.SemaphoreType.DMA,
            pltpu.SemaphoreType.DMA,
            pltpu.SemaphoreType.DMA,
        ],
    )
    return k(qn, kn, qt, kt, nsrc4, ndst4, tsrc4, tdst4, tndst4, ntdst4)


HALF_N = N_NODE // 2
HALF_T = N_TRI // 2
AC_N = 5120
AC_T = 10112


def _scb_body(vn, vt, nsrc4, ndst4, pn4, tsrc4, tdst4, pt4,
              accn_o, acct_o, sn_o, st_o,
              vrowsA, vrowsB, zbuf2, zbuf1, dstlocA, dstlocB,
              pblk, src_blk, dst_blk, acc_sh, s_sh, semA, semB, semSA, semSB):
    c = lax.axis_index("c")
    s = lax.axis_index("s")
    lane = lax.iota(jnp.int32, 16)

    def zfill(r, _):
        for jj in range(8):
            zbuf2[r, pl.ds(jj * 16, 16)] = jnp.zeros((16,), jnp.float32)
        return 0
    lax.fori_loop(0, 8, zfill, 0)

    def z1fill(i, _):
        zbuf1[pl.ds(i * 16, 16)] = jnp.zeros((16,), jnp.float32)
        return 0
    lax.fori_loop(0, 40, z1fill, 0)

    def zero_acc(rows_per_sub):
        def zcopy(q, _):
            pltpu.sync_copy(zbuf2, acc_sh.at[pl.ds(s * rows_per_sub + q * 8, 8)])
            return 0
        lax.fori_loop(0, rows_per_sub // 8, zcopy, 0)

    def zero_s(region):
        if region == AC_N:
            @pl.when(s < 8)
            def _():
                pltpu.sync_copy(zbuf1, s_sh.at[pl.ds(s * 640, 640)])
        else:
            @pl.when(s < 15)
            def _():
                pltpu.sync_copy(zbuf1, s_sh.at[pl.ds(s * 640, 640)])

            @pl.when(s == 15)
            def _():
                pltpu.sync_copy(zbuf1.at[pl.ds(0, 512)], s_sh.at[pl.ds(9600, 512)])

    def bjob(v_hbm, src4_hbm, dst4_hbm, p4_hbm, half, dummy):
        base = c * half

        def compute(row, vrows, dstloc):
            def group(g, _):
                dv = dst_blk[row, pl.ds(g * 16, 16)]
                loc = dv - base
                ok = (loc >= 0) & (loc < half)
                dstloc[pl.ds(g * 16, 16)] = jnp.where(ok, loc, dummy)
                pchunk = pblk[row, pl.ds(g * 16, 16)]

                def edge16(i):
                    e = g * 16 + i
                    sc = jnp.sum(jnp.where(lane == i, pchunk, 0.0))
                    pv = jnp.full((16,), sc, jnp.float32)
                    for jj in range(8):
                        vrows[e, pl.ds(jj * 16, 16)] = vrows[e, pl.ds(jj * 16, 16)] * pv
                plsc.parallel_loop(0, 16, unroll=2)(edge16)
                return 0
            lax.fori_loop(0, CH // 16, group, 0)

        def super_it(t, _):
            pltpu.sync_copy(src4_hbm.at[s, t], src_blk)
            pltpu.sync_copy(dst4_hbm.at[s, t], dst_blk)
            pltpu.sync_copy(p4_hbm.at[s, t], pblk)
            for jp in range(BLKB // 2):
                ra, rb = 2 * jp, 2 * jp + 1
                if jp > 0:
                    pltpu.make_async_copy(vrowsA, acc_sh.at[dstlocA], semSA).wait()
                else:
                    @pl.when(t > 0)
                    def _():
                        pltpu.make_async_copy(vrowsA, acc_sh.at[dstlocA], semSA).wait()
                cpa = pltpu.async_copy(v_hbm.at[src_blk.at[ra]], vrowsA, semA)
                if jp > 0:
                    pltpu.make_async_copy(vrowsB, acc_sh.at[dstlocB], semSB).wait()
                else:
                    @pl.when(t > 0)
                    def _():
                        pltpu.make_async_copy(vrowsB, acc_sh.at[dstlocB], semSB).wait()
                cpb = pltpu.async_copy(v_hbm.at[src_blk.at[rb]], vrowsB, semB)
                cpa.wait()
                compute(ra, vrowsA, dstlocA)
                pltpu.sync_copy(pblk.at[ra], s_sh.at[dstlocA], add=True)
                pltpu.async_copy(vrowsA, acc_sh.at[dstlocA], semSA, add=True)
                cpb.wait()
                compute(rb, vrowsB, dstlocB)
                pltpu.sync_copy(pblk.at[rb], s_sh.at[dstlocB], add=True)
                pltpu.async_copy(vrowsB, acc_sh.at[dstlocB], semSB, add=True)
            return 0
        lax.fori_loop(0, NBLK_B, super_it, 0)
        pltpu.make_async_copy(vrowsA, acc_sh.at[dstlocA], semSA).wait()
        pltpu.make_async_copy(vrowsB, acc_sh.at[dstlocB], semSB).wait()

    zero_acc(AC_T // NS)
    zero_s(AC_T)
    plsc.subcore_barrier()
    bjob(vn, nsrc4, ndst4, pn4, HALF_N, HALF_N)
    plsc.subcore_barrier()
    pltpu.sync_copy(acc_sh.at[pl.ds(s * 320, 320)], accn_o.at[c, pl.ds(s * 320, 320)])

    @pl.when(s < 8)
    def _():
        pltpu.sync_copy(s_sh.at[pl.ds(s * 640, 640)], sn_o.at[c, 0, pl.ds(s * 640, 640)])
    plsc.subcore_barrier()

    zero_acc(AC_N // NS)
    zero_s(AC_N)
    plsc.subcore_barrier()
    bjob(vt, tsrc4, tdst4, pt4, HALF_T, HALF_T)
    plsc.subcore_barrier()
    pltpu.sync_copy(acc_sh.at[pl.ds(s * 632, 632)], acct_o.at[c, pl.ds(s * 632, 632)])

    @pl.when(s < 15)
    def _():
        pltpu.sync_copy(s_sh.at[pl.ds(s * 640, 640)], st_o.at[c, 0, pl.ds(s * 640, 640)])

    @pl.when(s == 15)
    def _():
        pltpu.sync_copy(s_sh.at[pl.ds(9600, 512)], st_o.at[c, 0, pl.ds(9600, 512)])


def _scb(vn, vt, nsrc4, ndst4, pn4, tsrc4, tdst4, pt4):
    mesh = plsc.VectorSubcoreMesh(core_axis_name="c", subcore_axis_name="s")
    k = pl.kernel(
        _scb_body, mesh=mesh,
        compiler_params=pltpu.CompilerParams(needs_layout_passes=False),
        out_type=[jax.ShapeDtypeStruct((NC, AC_N, D), jnp.float32),
                  jax.ShapeDtypeStruct((NC, AC_T, D), jnp.float32),
                  jax.ShapeDtypeStruct((NC, 1, AC_N), jnp.float32),
                  jax.ShapeDtypeStruct((NC, 1, AC_T), jnp.float32)],
        scratch_types=[
            pltpu.VMEM((CH, D), jnp.float32),
            pltpu.VMEM((CH, D), jnp.float32),
            pltpu.VMEM((8, D), jnp.float32),
            pltpu.VMEM((640,), jnp.float32),
            pltpu.VMEM((CH,), jnp.int32),
            pltpu.VMEM((CH,), jnp.int32),
            pltpu.VMEM((BLKB, CH), jnp.float32),
            pltpu.VMEM((BLKB, CH), jnp.int32),
            pltpu.VMEM((BLKB, CH), jnp.int32),
            pltpu.VMEM_SHARED((AC_T, D), jnp.float32),
            pltpu.VMEM_SHARED((AC_T,), jnp.float32),
            pltpu.SemaphoreType.DMA,
            pltpu.SemaphoreType.DMA,
            pltpu.SemaphoreType.DMA,
            pltpu.SemaphoreType.DMA,
        ],
    )
    return k(vn, vt, nsrc4, ndst4, pn4, tsrc4, tdst4, pt4)


def _epi_body(orig_ref, acc_ref, s_ref, msk_ref, gc_ref, bc_ref, vw_ref, vb_ref,
              ow_ref, ob_ref, gm_ref, bm_ref, w1_ref, b1_ref, w2_ref, b2_ref,
              out_ref):
    sa = acc_ref[...] / (s_ref[...] + 1e-16)
    x1 = orig_ref[...] + sa
    xn = _ln(x1, gc_ref[...], bc_ref[...])
    v = jnp.dot(xn, vw_ref[...], preferred_element_type=jnp.float32) + vb_ref[...]
    v = jnp.where(msk_ref[...] > 0, v, 0.0)
    xnew = jnp.dot(v, ow_ref[...], preferred_element_type=jnp.float32) + ob_ref[...] + xn
    x2 = x1 + xnew
    xm = _ln(x2, gm_ref[...], bm_ref[...])
    h = jax.nn.gelu(jnp.dot(xm, w1_ref[...], preferred_element_type=jnp.float32) + b1_ref[...])
    out_ref[...] = x2 + jnp.dot(h, w2_ref[...], preferred_element_type=jnp.float32) + b2_ref[...]


def _epi(orig, acc, sb, cb, gc, bc, pc, gm, bm, pm, rb):
    n = orig.shape[0]
    row = lambda i: (i, 0)
    zero = lambda i: (0, 0)
    full = lambda s: pl.BlockSpec(s, zero)
    return pl.pallas_call(
        _epi_body,
        grid=(n // rb,),
        in_specs=[pl.BlockSpec((rb, D), row), pl.BlockSpec((rb, D), row),
                  pl.BlockSpec((rb, D), row), pl.BlockSpec((rb, D), row),
                  full((1, D)), full((1, D)), full((D, D)), full((1, D)),
                  full((D, D)), full((1, D)), full((1, D)), full((1, D)),
                  full((D, 4 * D)), full((1, 4 * D)), full((4 * D, D)), full((1, D))],
        out_specs=pl.BlockSpec((rb, D), row),
        out_shape=jax.ShapeDtypeStruct((n, D), jnp.float32),
    )(orig, acc, sb, cb, gc.reshape(1, D), bc.reshape(1, D),
      pc['v_w'], pc['v_b'].reshape(1, D), pc['out_w'], pc['out_b'].reshape(1, D),
      gm.reshape(1, D), bm.reshape(1, D),
      pm['w1'], pm['b1'].reshape(1, 4 * D), pm['w2'], pm['b2'].reshape(1, D))


def kernel(node, triangle, params, node_edge_index, tri_edge_index, tn_edge_index, nt_edge_index):
    P = params
    ln = P['ln']

    nsrcA = node_edge_index[0].reshape(W, NBLK_A, BLKA, CH)
    ndstA = node_edge_index[1].reshape(W, NBLK_A, BLKA, CH)
    tsrcA = tri_edge_index[0].reshape(W, NBLK_A, BLKA, CH)
    tdstA = tri_edge_index[1].reshape(W, NBLK_A, BLKA, CH)
    nsrcB = node_edge_index[0].reshape(NS, NBLK_B, BLKB, CH)
    ndstB = node_edge_index[1].reshape(NS, NBLK_B, BLKB, CH)
    tsrcB = tri_edge_index[0].reshape(NS, NBLK_B, BLKB, CH)
    tdstB = tri_edge_index[1].reshape(NS, NBLK_B, BLKB, CH)
    tndst4 = tn_edge_index[1].reshape(CW, 5, BLKA, CH)
    ntdst4 = nt_edge_index[1].reshape(CW, 5, BLKA, CH)

    qn, kn, vn = _proj(node, ln['node_self_g'], ln['node_self_b'], P['node_sa'], 1000)
    qt, kt, vt = _proj(triangle, ln['tri_self_g'], ln['tri_self_b'], P['tri_sa'], 1000)

    pn4, pt4, ctn2, cnt2 = _sca(qn, kn, qt, kt, nsrcA, ndstA, tsrcA, tdstA, tndst4, ntdst4)

    accn, acct, sn2, st2 = _scb(vn, vt, nsrcB, ndstB, pn4.reshape(NS, NBLK_B, BLKB, CH),
                                tsrcB, tdstB, pt4.reshape(NS, NBLK_B, BLKB, CH))
    accn_full = jnp.concatenate([accn[0, :HALF_N], accn[1, :HALF_N]], axis=0)
    acct_full = jnp.concatenate([acct[0, :HALF_T], acct[1, :HALF_T]], axis=0)

    sn = jnp.concatenate([sn2[0, 0, :HALF_N], sn2[1, 0, :HALF_N]])
    st = jnp.concatenate([st2[0, 0, :HALF_T], st2[1, 0, :HALF_T]])
    ctn = ctn2[0, 0, :N_NODE] + ctn2[1, 0, :N_NODE]
    cnt = cnt2[0, 0, :N_TRI] + cnt2[1, 0, :N_TRI]

    node_out = _epi(node, accn_full,
                    jnp.broadcast_to(sn[:, None], (N_NODE, D)),
                    jnp.broadcast_to(ctn[:, None], (N_NODE, D)),
                    ln['node_cross_g'], ln['node_cross_b'], P['tri2node'],
                    ln['node_mlp_g'], ln['node_mlp_b'], P['mlp_node'], 1000)
    tri_out = _epi(triangle, acct_full,
                   jnp.broadcast_to(st[:, None], (N_TRI, D)),
                   jnp.broadcast_to(cnt[:, None], (N_TRI, D)),
                   ln['tri_cross_g'], ln['tri_cross_b'], P['node2tri'],
                   ln['tri_mlp_g'], ln['tri_mlp_b'], P['mlp_tri'], 1000)
    return node_out, tri_out
